# Initial kernel scaffold; baseline (speedup 1.0000x reference)
import jax
import jax.numpy as jnp
from jax.experimental import pallas as pl


def kernel(pred_objectness_logits, pred_deltas):
    raise NotImplementedError("write your pallas kernel here")


# speedup vs baseline: 7.0409x; 7.0409x over previous
import functools

import jax
import jax.numpy as jnp
from jax.experimental import pallas as pl
from jax.experimental.pallas import tpu as pltpu

STRIDE = 4
NMS_THRESH = 0.7
PRE_NMS_TOPK = 2000
POST_NMS_TOPK = 1000
BIG_NEG = -1e9

NPAD = 2048
BLK = 128
NBLK = NPAD // BLK


def _nms_kernel(dr_ref, cr_ref, dc_ref, cc_ref, keep_ref, boxes_ref, s_ref):
    cx = cr_ref[0, 0:1, :]
    cy = cr_ref[0, 1:2, :]
    x1 = cx + dr_ref[0, 0:1, :] * float(STRIDE)
    y1 = cy + dr_ref[0, 1:2, :] * float(STRIDE)
    x2 = cx + dr_ref[0, 2:3, :] * float(STRIDE)
    y2 = cy + dr_ref[0, 3:4, :] * float(STRIDE)
    bx1 = jnp.minimum(x1, x2)
    bx2 = jnp.maximum(x1, x2)
    by1 = jnp.minimum(y1, y2)
    by2 = jnp.maximum(y1, y2)
    area = jnp.maximum(bx2 - bx1, 0.0) * jnp.maximum(by2 - by1, 0.0)

    boxes_ref[0, 0:1, :] = bx1
    boxes_ref[0, 1:2, :] = by1
    boxes_ref[0, 2:3, :] = bx2
    boxes_ref[0, 3:4, :] = by2

    keep_ref[0] = jnp.ones((1, NPAD), jnp.float32)

    lane128 = jax.lax.broadcasted_iota(jnp.int32, (1, BLK), 1)
    lane_all = jax.lax.broadcasted_iota(jnp.int32, (1, NPAD), 1)

    for bj in range(NBLK):
        s = bj * BLK
        ccb = cc_ref[0, s:s + BLK, :]
        dcb = dc_ref[0, s:s + BLK, :]
        cxT = ccb[:, 0:1]
        cyT = ccb[:, 1:2]
        x1T = cxT + dcb[:, 0:1] * float(STRIDE)
        y1T = cyT + dcb[:, 1:2] * float(STRIDE)
        x2T = cxT + dcb[:, 2:3] * float(STRIDE)
        y2T = cyT + dcb[:, 3:4] * float(STRIDE)
        bx1T = jnp.minimum(x1T, x2T)
        bx2T = jnp.maximum(x1T, x2T)
        by1T = jnp.minimum(y1T, y2T)
        by2T = jnp.maximum(y1T, y2T)
        areaT = jnp.maximum(bx2T - bx1T, 0.0) * jnp.maximum(by2T - by1T, 0.0)

        xx1 = jnp.maximum(bx1T, bx1)
        yy1 = jnp.maximum(by1T, by1)
        xx2 = jnp.minimum(bx2T, bx2)
        yy2 = jnp.minimum(by2T, by2)
        inter = jnp.maximum(xx2 - xx1, 0.0) * jnp.maximum(yy2 - yy1, 0.0)
        union = areaT + area - inter
        iou = inter / jnp.maximum(union, 1e-6)
        sup = (iou > NMS_THRESH).astype(jnp.float32)
        s_ref[...] = sup

        kb = keep_ref[0, :, s:s + BLK]
        sub8 = jax.lax.broadcasted_iota(jnp.int32, (8, BLK), 0)
        for k in range(BLK // 8):
            chunk = s_ref[8 * k:8 * k + 8, s:s + BLK]

            def body(j, kb, k=k, chunk=chunk):
                i = 8 * k + j
                row = jnp.sum(jnp.where(sub8 == j, chunk, 0.0), axis=0,
                              keepdims=True)
                ki = jnp.sum(jnp.where(lane128 == i, kb, 0.0))
                later = (lane128 > i).astype(jnp.float32)
                return kb * (1.0 - ki * row * later)

            kb = jax.lax.fori_loop(0, 8, body, kb)
        keep_ref[0, :, s:s + BLK] = kb

        if bj < NBLK - 1:
            supv = jnp.dot(kb, s_ref[...], preferred_element_type=jnp.float32)
            hit = ((supv > 0.5) & (lane_all >= s + BLK)).astype(jnp.float32)
            keep_ref[0] = keep_ref[0] * (1.0 - hit)


@jax.jit
def kernel(pred_objectness_logits, pred_deltas):
    B, _, H, W = pred_objectness_logits.shape
    HW = H * W
    logits = pred_objectness_logits.reshape(B, HW)
    deltas = pred_deltas.reshape(B, 4, HW)

    top_scores, top_idx = jax.lax.top_k(logits, PRE_NMS_TOPK)

    pad = NPAD - PRE_NMS_TOPK
    idx_p = jnp.pad(top_idx, ((0, 0), (0, pad)))
    dg = jnp.take_along_axis(deltas, idx_p[:, None, :], axis=2)
    dg = dg * (jnp.arange(NPAD) < PRE_NMS_TOPK).astype(dg.dtype)
    cx = (idx_p % W).astype(jnp.float32) * float(STRIDE)
    cy = (idx_p // W).astype(jnp.float32) * float(STRIDE)
    cg = jnp.stack([cx, cy], axis=1)
    cg = cg * (jnp.arange(NPAD) < PRE_NMS_TOPK).astype(cg.dtype)

    dgT = jnp.transpose(dg, (0, 2, 1))
    cgT = jnp.transpose(cg, (0, 2, 1))

    keep_f, boxes_r = pl.pallas_call(
        _nms_kernel,
        grid=(B,),
        in_specs=[
            pl.BlockSpec((1, 4, NPAD), lambda b: (b, 0, 0)),
            pl.BlockSpec((1, 2, NPAD), lambda b: (b, 0, 0)),
            pl.BlockSpec((1, NPAD, 4), lambda b: (b, 0, 0)),
            pl.BlockSpec((1, NPAD, 2), lambda b: (b, 0, 0)),
        ],
        out_specs=[
            pl.BlockSpec((1, 1, NPAD), lambda b: (b, 0, 0)),
            pl.BlockSpec((1, 4, NPAD), lambda b: (b, 0, 0)),
        ],
        out_shape=[
            jax.ShapeDtypeStruct((B, 1, NPAD), jnp.float32),
            jax.ShapeDtypeStruct((B, 4, NPAD), jnp.float32),
        ],
        scratch_shapes=[pltpu.VMEM((BLK, NPAD), jnp.float32)],
    )(dg, cg, dgT, cgT)

    keep = keep_f[:, 0, :PRE_NMS_TOPK] > 0.5
    top_boxes = jnp.transpose(boxes_r[:, :, :PRE_NMS_TOPK], (0, 2, 1))

    def select(keep_i, boxes_i, scores_i):
        order = jnp.argsort(jnp.where(keep_i, 0, 1))
        sel = order[:POST_NMS_TOPK]
        kept = keep_i[sel]
        out_boxes = boxes_i[sel]
        out_scores = jnp.where(kept, scores_i[sel], BIG_NEG)
        return jnp.concatenate([out_boxes, out_scores[:, None]], axis=-1)

    return jax.vmap(select)(keep, top_boxes, top_scores)


# speedup vs baseline: 10.5102x; 1.4927x over previous
import jax
import jax.numpy as jnp
from jax.experimental import pallas as pl
from jax.experimental.pallas import tpu as pltpu

STRIDE = 4
NMS_THRESH = 0.7
PRE_NMS_TOPK = 2000
POST_NMS_TOPK = 1000
BIG_NEG = -1e9

NPAD = 2048
BLK = 128
NBLK = NPAD // BLK
NIMG = 2


def _nms_kernel(dr_ref, cr_ref, dc_ref, cc_ref, keep_ref, boxes_ref, s_ref):
    lane128 = jax.lax.broadcasted_iota(jnp.int32, (1, BLK), 1)
    lane_all = jax.lax.broadcasted_iota(jnp.int32, (1, NPAD), 1)
    sub8 = jax.lax.broadcasted_iota(jnp.int32, (8, BLK), 0)

    rows = []
    for a in range(NIMG):
        cx = cr_ref[a, 0:1, :]
        cy = cr_ref[a, 1:2, :]
        x1 = cx + dr_ref[a, 0:1, :] * float(STRIDE)
        y1 = cy + dr_ref[a, 1:2, :] * float(STRIDE)
        x2 = cx + dr_ref[a, 2:3, :] * float(STRIDE)
        y2 = cy + dr_ref[a, 3:4, :] * float(STRIDE)
        bx1 = jnp.minimum(x1, x2)
        bx2 = jnp.maximum(x1, x2)
        by1 = jnp.minimum(y1, y2)
        by2 = jnp.maximum(y1, y2)
        area = jnp.maximum(bx2 - bx1, 0.0) * jnp.maximum(by2 - by1, 0.0)
        boxes_ref[a, 0:1, :] = bx1
        boxes_ref[a, 1:2, :] = by1
        boxes_ref[a, 2:3, :] = bx2
        boxes_ref[a, 3:4, :] = by2
        rows.append((bx1, by1, bx2, by2, area))

    keep_ref[...] = jnp.ones((NIMG, 1, NPAD), jnp.float32)

    for bj in range(NBLK):
        s = bj * BLK
        for a in range(NIMG):
            bx1, by1, bx2, by2, area = rows[a]
            ccb = cc_ref[a, s:s + BLK, :]
            dcb = dc_ref[a, s:s + BLK, :]
            cxT = ccb[:, 0:1]
            cyT = ccb[:, 1:2]
            x1T = cxT + dcb[:, 0:1] * float(STRIDE)
            y1T = cyT + dcb[:, 1:2] * float(STRIDE)
            x2T = cxT + dcb[:, 2:3] * float(STRIDE)
            y2T = cyT + dcb[:, 3:4] * float(STRIDE)
            bx1T = jnp.minimum(x1T, x2T)
            bx2T = jnp.maximum(x1T, x2T)
            by1T = jnp.minimum(y1T, y2T)
            by2T = jnp.maximum(y1T, y2T)
            areaT = (jnp.maximum(bx2T - bx1T, 0.0) *
                     jnp.maximum(by2T - by1T, 0.0))
            inter = (jnp.maximum(jnp.minimum(bx2T, bx2) -
                                 jnp.maximum(bx1T, bx1), 0.0) *
                     jnp.maximum(jnp.minimum(by2T, by2) -
                                 jnp.maximum(by1T, by1), 0.0))
            union = areaT + area - inter
            iou = inter / jnp.maximum(union, 1e-6)
            s_ref[a * BLK:(a + 1) * BLK, :] = (
                (iou > NMS_THRESH).astype(jnp.float32))

        kb = keep_ref[:, 0, s:s + BLK]
        for k in range(BLK // 8):
            chunks = [s_ref[a * BLK + 8 * k:a * BLK + 8 * k + 8, s:s + BLK]
                      for a in range(NIMG)]

            def body(j, kb, k=k, chunks=chunks):
                i = 8 * k + j
                r = jnp.concatenate(
                    [jnp.sum(jnp.where(sub8 == j, c, 0.0), axis=0,
                             keepdims=True) for c in chunks], axis=0)
                ki = jnp.sum(jnp.where(lane128 == i, kb, 0.0), axis=1,
                             keepdims=True)
                later = (lane128 > i).astype(jnp.float32)
                return kb * (1.0 - ki * r * later)

            kb = jax.lax.fori_loop(0, 8, body, kb)
        keep_ref[:, 0, s:s + BLK] = kb

        if bj < NBLK - 1:
            supv = jnp.concatenate(
                [jnp.dot(kb[a:a + 1, :], s_ref[a * BLK:(a + 1) * BLK, :],
                         preferred_element_type=jnp.float32)
                 for a in range(NIMG)], axis=0)
            hit = ((supv > 0.5) & (lane_all >= s + BLK)).astype(jnp.float32)
            keep_ref[:, 0, :] = keep_ref[:, 0, :] * (1.0 - hit)


@jax.jit
def kernel(pred_objectness_logits, pred_deltas):
    B, _, H, W = pred_objectness_logits.shape
    HW = H * W
    logits = pred_objectness_logits.reshape(B, HW)
    deltas = pred_deltas.reshape(B, 4, HW)

    top_scores, top_idx = jax.lax.top_k(logits, PRE_NMS_TOPK)

    pad = NPAD - PRE_NMS_TOPK
    idx_p = jnp.pad(top_idx, ((0, 0), (0, pad)))
    dg = jnp.take_along_axis(deltas, idx_p[:, None, :], axis=2)
    dg = dg * (jnp.arange(NPAD) < PRE_NMS_TOPK).astype(dg.dtype)
    cx = (idx_p % W).astype(jnp.float32) * float(STRIDE)
    cy = (idx_p // W).astype(jnp.float32) * float(STRIDE)
    cg = jnp.stack([cx, cy], axis=1)
    cg = cg * (jnp.arange(NPAD) < PRE_NMS_TOPK).astype(cg.dtype)

    dgT = jnp.transpose(dg, (0, 2, 1))
    cgT = jnp.transpose(cg, (0, 2, 1))

    keep_f, boxes_r = pl.pallas_call(
        _nms_kernel,
        out_shape=[
            jax.ShapeDtypeStruct((B, 1, NPAD), jnp.float32),
            jax.ShapeDtypeStruct((B, 4, NPAD), jnp.float32),
        ],
        scratch_shapes=[pltpu.VMEM((NIMG * BLK, NPAD), jnp.float32)],
    )(dg, cg, dgT, cgT)

    keep = keep_f[:, 0, :PRE_NMS_TOPK] > 0.5
    top_boxes = jnp.transpose(boxes_r[:, :, :PRE_NMS_TOPK], (0, 2, 1))

    def select(keep_i, boxes_i, scores_i):
        order = jnp.argsort(jnp.where(keep_i, 0, 1))
        sel = order[:POST_NMS_TOPK]
        kept = keep_i[sel]
        out_boxes = boxes_i[sel]
        out_scores = jnp.where(kept, scores_i[sel], BIG_NEG)
        return jnp.concatenate([out_boxes, out_scores[:, None]], axis=-1)

    return jax.vmap(select)(keep, top_boxes, top_scores)


# speedup vs baseline: 11.1656x; 1.0624x over previous
import jax
import jax.numpy as jnp
from jax.experimental import pallas as pl
from jax.experimental.pallas import tpu as pltpu

STRIDE = 4
NMS_THRESH = 0.7
PRE_NMS_TOPK = 2000
POST_NMS_TOPK = 1000
BIG_NEG = -1e9

NPAD = 2048
BLK = 128
NBLK = NPAD // BLK
NIMG = 2


def _nms_kernel(dr_ref, cr_ref, dc_ref, cc_ref, keep_ref, boxes_ref, s_ref):
    lane128 = jax.lax.broadcasted_iota(jnp.int32, (1, BLK), 1)
    lane_all = jax.lax.broadcasted_iota(jnp.int32, (1, NPAD), 1)
    sub8 = jax.lax.broadcasted_iota(jnp.int32, (8, BLK), 0)

    rows = []
    for a in range(NIMG):
        cx = cr_ref[a, 0:1, :]
        cy = cr_ref[a, 1:2, :]
        x1 = cx + dr_ref[a, 0:1, :] * float(STRIDE)
        y1 = cy + dr_ref[a, 1:2, :] * float(STRIDE)
        x2 = cx + dr_ref[a, 2:3, :] * float(STRIDE)
        y2 = cy + dr_ref[a, 3:4, :] * float(STRIDE)
        bx1 = jnp.minimum(x1, x2)
        bx2 = jnp.maximum(x1, x2)
        by1 = jnp.minimum(y1, y2)
        by2 = jnp.maximum(y1, y2)
        area = jnp.maximum(bx2 - bx1, 0.0) * jnp.maximum(by2 - by1, 0.0)
        boxes_ref[a, 0:1, :] = bx1
        boxes_ref[a, 1:2, :] = by1
        boxes_ref[a, 2:3, :] = bx2
        boxes_ref[a, 3:4, :] = by2
        rows.append((bx1, by1, bx2, by2, area))

    keep_ref[...] = jnp.ones((NIMG, 1, NPAD), jnp.float32)

    for bj in range(NBLK):
        s = bj * BLK
        R = NPAD - s
        rowio = jax.lax.broadcasted_iota(jnp.int32, (BLK, R), 0)
        laneio = jax.lax.broadcasted_iota(jnp.int32, (BLK, R), 1)
        allow = (laneio >= BLK) | (laneio > rowio)
        for a in range(NIMG):
            bx1, by1, bx2, by2, area = rows[a]
            ccb = cc_ref[a, s:s + BLK, :]
            dcb = dc_ref[a, s:s + BLK, :]
            cxT = ccb[:, 0:1]
            cyT = ccb[:, 1:2]
            x1T = cxT + dcb[:, 0:1] * float(STRIDE)
            y1T = cyT + dcb[:, 1:2] * float(STRIDE)
            x2T = cxT + dcb[:, 2:3] * float(STRIDE)
            y2T = cyT + dcb[:, 3:4] * float(STRIDE)
            bx1T = jnp.minimum(x1T, x2T)
            bx2T = jnp.maximum(x1T, x2T)
            by1T = jnp.minimum(y1T, y2T)
            by2T = jnp.maximum(y1T, y2T)
            areaT = (jnp.maximum(bx2T - bx1T, 0.0) *
                     jnp.maximum(by2T - by1T, 0.0))
            inter = (jnp.maximum(jnp.minimum(bx2T, bx2[:, s:]) -
                                 jnp.maximum(bx1T, bx1[:, s:]), 0.0) *
                     jnp.maximum(jnp.minimum(by2T, by2[:, s:]) -
                                 jnp.maximum(by1T, by1[:, s:]), 0.0))
            union = areaT + area[:, s:] - inter
            iou = inter / jnp.maximum(union, 1e-6)
            s_ref[a * BLK:(a + 1) * BLK, s:] = (
                ((iou > NMS_THRESH) & allow).astype(jnp.float32))

        kb = [keep_ref[a, :, s:s + BLK] for a in range(NIMG)]
        for k in range(BLK // 8):
            ch = [s_ref[a * BLK + 8 * k:a * BLK + 8 * k + 8, s:s + BLK]
                  for a in range(NIMG)]
            for j in range(8):
                i = 8 * k + j
                m8 = sub8 == j
                mi = lane128 == i
                for a in range(NIMG):
                    row = jnp.sum(jnp.where(m8, ch[a], 0.0), axis=0,
                                  keepdims=True)
                    ki = jnp.sum(jnp.where(mi, kb[a], 0.0), axis=1,
                                 keepdims=True)
                    kb[a] = kb[a] * (1.0 - ki * row)
        for a in range(NIMG):
            keep_ref[a, :, s:s + BLK] = kb[a]

        if bj < NBLK - 1:
            for a in range(NIMG):
                supv = jnp.dot(kb[a], s_ref[a * BLK:(a + 1) * BLK, s:],
                               preferred_element_type=jnp.float32)
                hit = (supv > 0.5).astype(jnp.float32)
                keep_ref[a, :, s:] = keep_ref[a, :, s:] * (1.0 - hit)


@jax.jit
def kernel(pred_objectness_logits, pred_deltas):
    B, _, H, W = pred_objectness_logits.shape
    HW = H * W
    logits = pred_objectness_logits.reshape(B, HW)
    deltas = pred_deltas.reshape(B, 4, HW)

    top_scores, top_idx = jax.lax.top_k(logits, PRE_NMS_TOPK)

    pad = NPAD - PRE_NMS_TOPK
    idx_p = jnp.pad(top_idx, ((0, 0), (0, pad)))
    dg = jnp.take_along_axis(deltas, idx_p[:, None, :], axis=2)
    dg = dg * (jnp.arange(NPAD) < PRE_NMS_TOPK).astype(dg.dtype)
    cx = (idx_p % W).astype(jnp.float32) * float(STRIDE)
    cy = (idx_p // W).astype(jnp.float32) * float(STRIDE)
    cg = jnp.stack([cx, cy], axis=1)
    cg = cg * (jnp.arange(NPAD) < PRE_NMS_TOPK).astype(cg.dtype)

    dgT = jnp.transpose(dg, (0, 2, 1))
    cgT = jnp.transpose(cg, (0, 2, 1))

    keep_f, boxes_r = pl.pallas_call(
        _nms_kernel,
        out_shape=[
            jax.ShapeDtypeStruct((B, 1, NPAD), jnp.float32),
            jax.ShapeDtypeStruct((B, 4, NPAD), jnp.float32),
        ],
        scratch_shapes=[pltpu.VMEM((NIMG * BLK, NPAD), jnp.float32)],
    )(dg, cg, dgT, cgT)

    keep = keep_f[:, 0, :PRE_NMS_TOPK] > 0.5
    top_boxes = jnp.transpose(boxes_r[:, :, :PRE_NMS_TOPK], (0, 2, 1))

    def select(keep_i, boxes_i, scores_i):
        order = jnp.argsort(jnp.where(keep_i, 0, 1))
        sel = order[:POST_NMS_TOPK]
        kept = keep_i[sel]
        out_boxes = boxes_i[sel]
        out_scores = jnp.where(kept, scores_i[sel], BIG_NEG)
        return jnp.concatenate([out_boxes, out_scores[:, None]], axis=-1)

    return jax.vmap(select)(keep, top_boxes, top_scores)
